# baseline (speedup 1.0000x reference)
import functools

import jax
import jax.numpy as jnp
from jax import lax
from jax.experimental import pallas as pl
from jax.experimental.pallas import tpu as pltpu, tpu_sc as plsc

N = 10000
E = 320000
D = 128
NC = 2
NS = 16
NW = NC * NS
CH = 128
KS = 16
EPW = E // NW
TAIL = EPW - (EPW // CH) * CH
STAGES = (16, 16, 16, 16, 14)
RPT = 640
NPAD = NS * RPT

_mesh = plsc.VectorSubcoreMesh(core_axis_name="c", subcore_axis_name="s")


@functools.partial(
    pl.kernel,
    out_type=(
        jax.ShapeDtypeStruct((NC, NPAD, D), jnp.float32),
        jax.ShapeDtypeStruct((NW, NPAD // 128, 128), jnp.float32),
    ),
    mesh=_mesh,
    compiler_params=pltpu.CompilerParams(needs_layout_passes=False),
    scratch_types=[
        pltpu.VMEM((KS, CH), jnp.int32),
        pltpu.VMEM((KS, CH), jnp.int32),
        pltpu.VMEM((CH, D), jnp.float32),
        pltpu.VMEM((CH, D), jnp.float32),
        pltpu.VMEM((NPAD // 128, 128), jnp.float32),
        pltpu.VMEM_SHARED((NPAD, D), jnp.float32),
        pltpu.SemaphoreType.DMA,
        pltpu.SemaphoreType.DMA,
        pltpu.SemaphoreType.DMA,
    ],
)
def _sc_aggregate(feat_hbm, srce_hbm, dste_hbm, zeros_hbm, out_hbm, deg_hbm,
                  src_v, dst_v, rows0, rows1, deg_v, agg_sh, sem0, sem1, sems):
    cid = lax.axis_index("c")
    sid = lax.axis_index("s")
    w = cid * NS + sid
    r0 = sid * RPT
    ebase = w * EPW

    zero16 = jnp.zeros((16,), jnp.float32)
    one16 = jnp.ones((16,), jnp.float32)

    def zbody(i, carry):
        deg_v[i // 8, pl.ds((i % 8) * 16, 16)] = zero16
        return carry

    def _wait(sem, buf):
        pltpu.make_async_copy(feat_hbm.at[src_v.at[0]], buf, sem).wait()

    def _stage(st, nch, tail):
        sbase = ebase + st * (KS * CH)
        for k in range(nch):
            pltpu.async_copy(srce_hbm.at[pl.ds(sbase + k * CH, CH)],
                             src_v.at[k], sems)
            pltpu.async_copy(dste_hbm.at[pl.ds(sbase + k * CH, CH)],
                             dst_v.at[k], sems)
        if tail:
            pltpu.async_copy(srce_hbm.at[pl.ds(ebase + EPW - CH, CH)],
                             src_v.at[nch], sems)
            pltpu.async_copy(dste_hbm.at[pl.ds(ebase + EPW - CH, CH)],
                             dst_v.at[nch], sems)
        nrows = nch + (1 if tail else 0)
        for k in range(2 * nrows):
            pltpu.make_async_copy(srce_hbm.at[pl.ds(sbase, CH)],
                                  src_v.at[0], sems).wait()

    for st, nch in enumerate(STAGES):
        tail = st == len(STAGES) - 1
        with jax.named_scope("stagecopy"):
            _stage(st, nch, tail)

        pltpu.async_copy(feat_hbm.at[src_v.at[0]], rows0, sem0)
        pltpu.async_copy(feat_hbm.at[src_v.at[1]], rows1, sem1)

        if st == 0:
            with jax.named_scope("initzero"):
                pltpu.sync_copy(zeros_hbm.at[pl.ds(r0, RPT)],
                                agg_sh.at[pl.ds(r0, RPT)])
                lax.fori_loop(0, NPAD // 16, zbody, 0)
            plsc.subcore_barrier()

        if tail:
            zero16i = jnp.zeros((16,), jnp.int32)
            iota16 = lax.iota(jnp.int32, 16)
            for g in range((CH - TAIL) // 16):
                src_v[nch, pl.ds(g * 16, 16)] = zero16i
                dst_v[nch, pl.ds(g * 16, 16)] = N + g * 16 + iota16

        ncheff = nch + (1 if tail else 0)
        nedges = ncheff * CH

        def dbody(i, carry):
            idx = dst_v[i // 8, pl.ds((i % 8) * 16, 16)]
            plsc.addupdate_scatter(deg_v, [idx >> 7, idx & 127], one16)
            return carry

        with jax.named_scope("degloop"):
            lax.fori_loop(0, nedges // 16, dbody, 0)

        def body(j, carry):
            c = 2 * j
            _wait(sem0, rows0)
            pltpu.sync_copy(rows0, agg_sh.at[dst_v.at[c]], add=True)
            pltpu.async_copy(feat_hbm.at[src_v.at[c + 2]], rows0, sem0)
            _wait(sem1, rows1)
            pltpu.sync_copy(rows1, agg_sh.at[dst_v.at[c + 1]], add=True)
            pltpu.async_copy(feat_hbm.at[src_v.at[c + 3]], rows1, sem1)
            return carry

        nloop = (ncheff - 2) // 2
        with jax.named_scope("gsloop"):
            lax.fori_loop(0, nloop, body, 0)
        c0 = 2 * nloop
        _wait(sem0, rows0)
        pltpu.sync_copy(rows0, agg_sh.at[dst_v.at[c0]], add=True)
        if ncheff % 2:
            pltpu.async_copy(feat_hbm.at[src_v.at[c0 + 2]], rows0, sem0)
        _wait(sem1, rows1)
        pltpu.sync_copy(rows1, agg_sh.at[dst_v.at[c0 + 1]], add=True)
        if ncheff % 2:
            _wait(sem0, rows0)
            pltpu.sync_copy(rows0, agg_sh.at[dst_v.at[c0 + 2]], add=True)

    with jax.named_scope("writeback"):
        pltpu.sync_copy(deg_v, deg_hbm.at[w])

        plsc.subcore_barrier()
        pltpu.sync_copy(agg_sh.at[pl.ds(r0, RPT)],
                        out_hbm.at[cid, pl.ds(r0, RPT)])


def _combine_body(feat_ref, agg_ref, deg_ref, wn_ref, ws_ref, b_ref, out_ref):
    x = feat_ref[...]
    neigh = agg_ref[0] + agg_ref[1]
    deg = jnp.sum(deg_ref[...], axis=0)[:, None]
    h = lax.dot_general(x, ws_ref[...], (((1,), (1,)), ((), ())),
                        preferred_element_type=jnp.float32)
    nb = lax.dot_general(neigh, wn_ref[...], (((1,), (1,)), ((), ())),
                         preferred_element_type=jnp.float32)
    out_ref[...] = h + b_ref[...] + nb * (1.0 / deg)


_BM = 1024


@jax.jit
def kernel(feat, edge_index, W_neigh, W_self, b_self):
    zeros = jnp.zeros((NPAD, D), jnp.float32)

    aggout, degout = _sc_aggregate(feat, edge_index[0], edge_index[1], zeros)
    degout = degout.reshape(NW, NPAD)

    rst = pl.pallas_call(
        _combine_body,
        grid=(pl.cdiv(N, _BM),),
        in_specs=[
            pl.BlockSpec((_BM, D), lambda i: (i, 0)),
            pl.BlockSpec((NC, _BM, D), lambda i: (0, i, 0)),
            pl.BlockSpec((NW, _BM), lambda i: (0, i)),
            pl.BlockSpec((D, D), lambda i: (0, 0)),
            pl.BlockSpec((D, D), lambda i: (0, 0)),
            pl.BlockSpec((1, D), lambda i: (0, 0)),
        ],
        out_specs=pl.BlockSpec((_BM, D), lambda i: (i, 0)),
        out_shape=jax.ShapeDtypeStruct((N, D), jnp.float32),
    )(feat, aggout, degout, W_neigh, W_self, b_self.reshape(1, D))
    return rst


# speedup vs baseline: 1.7410x; 1.7410x over previous
import functools

import jax
import jax.numpy as jnp
from jax import lax
from jax.experimental import pallas as pl
from jax.experimental.pallas import tpu as pltpu, tpu_sc as plsc

N = 10000
E = 320000
D = 128
NC = 2
NS = 16
NW = NC * NS
CH = 128
KS = 16
ST = 5
KCH = KS * ST
EPW = KCH * CH
EPAD = NW * EPW
RPT = 640
NPAD = NS * RPT

_mesh = plsc.VectorSubcoreMesh(core_axis_name="c", subcore_axis_name="s")


@functools.partial(
    pl.kernel,
    out_type=(
        jax.ShapeDtypeStruct((NC, NPAD, D), jnp.float32),
        jax.ShapeDtypeStruct((NW, NPAD // 128, 128), jnp.float32),
    ),
    mesh=_mesh,
    compiler_params=pltpu.CompilerParams(needs_layout_passes=False),
    scratch_types=[
        pltpu.VMEM((KS, CH), jnp.int32),
        pltpu.VMEM((KS, CH), jnp.int32),
        pltpu.VMEM((CH, D), jnp.float32),
        pltpu.VMEM((CH, D), jnp.float32),
        pltpu.VMEM((NPAD // 128, 128), jnp.float32),
        pltpu.VMEM_SHARED((NPAD, D), jnp.float32),
        pltpu.SemaphoreType.DMA,
        pltpu.SemaphoreType.DMA,
    ],
)
def _sc_aggregate(feat_hbm, srcv_hbm, dstv_hbm, zeros_hbm, out_hbm, deg_hbm,
                  src_v, dst_v, rows0, rows1, deg_v, agg_sh, sem0, sem1):
    cid = lax.axis_index("c")
    sid = lax.axis_index("s")
    w = cid * NS + sid
    r0 = sid * RPT

    zero16 = jnp.zeros((16,), jnp.float32)
    one16 = jnp.ones((16,), jnp.float32)

    def zbody(i, carry):
        deg_v[i // 8, pl.ds((i % 8) * 16, 16)] = zero16
        return carry

    def _wait(sem, buf):
        pltpu.make_async_copy(feat_hbm.at[src_v.at[0]], buf, sem).wait()

    for st in range(ST):
        with jax.named_scope("stagecopy"):
            pltpu.sync_copy(srcv_hbm.at[w, pl.ds(st * KS, KS)], src_v)
            pltpu.sync_copy(dstv_hbm.at[w, pl.ds(st * KS, KS)], dst_v)

        pltpu.async_copy(feat_hbm.at[src_v.at[0]], rows0, sem0)
        pltpu.async_copy(feat_hbm.at[src_v.at[1]], rows1, sem1)

        if st == 0:
            with jax.named_scope("initzero"):
                pltpu.sync_copy(zeros_hbm.at[pl.ds(r0, RPT)],
                                agg_sh.at[pl.ds(r0, RPT)])
                lax.fori_loop(0, NPAD // 16, zbody, 0)
            plsc.subcore_barrier()

        def dbody(i, carry):
            idx = dst_v[i // 8, pl.ds((i % 8) * 16, 16)]
            plsc.addupdate_scatter(deg_v, [idx >> 7, idx & 127], one16)
            return carry

        with jax.named_scope("degloop"):
            lax.fori_loop(0, (KS * CH) // 16, dbody, 0)

        def body(j, carry):
            c = 2 * j
            _wait(sem0, rows0)
            pltpu.sync_copy(rows0, agg_sh.at[dst_v.at[c]], add=True)
            pltpu.async_copy(feat_hbm.at[src_v.at[c + 2]], rows0, sem0)
            _wait(sem1, rows1)
            pltpu.sync_copy(rows1, agg_sh.at[dst_v.at[c + 1]], add=True)
            pltpu.async_copy(feat_hbm.at[src_v.at[c + 3]], rows1, sem1)
            return carry

        with jax.named_scope("gsloop"):
            lax.fori_loop(0, (KS - 2) // 2, body, 0)
        _wait(sem0, rows0)
        pltpu.sync_copy(rows0, agg_sh.at[dst_v.at[KS - 2]], add=True)
        _wait(sem1, rows1)
        pltpu.sync_copy(rows1, agg_sh.at[dst_v.at[KS - 1]], add=True)

    with jax.named_scope("writeback"):
        pltpu.sync_copy(deg_v, deg_hbm.at[w])

        plsc.subcore_barrier()
        pltpu.sync_copy(agg_sh.at[pl.ds(r0, RPT)],
                        out_hbm.at[cid, pl.ds(r0, RPT)])


def _combine_body(feat_ref, agg_ref, deg_ref, wn_ref, ws_ref, b_ref, out_ref):
    x = feat_ref[...]
    neigh = agg_ref[0] + agg_ref[1]
    deg = jnp.sum(deg_ref[...], axis=0)[:, None]
    h = lax.dot_general(x, ws_ref[...], (((1,), (1,)), ((), ())),
                        preferred_element_type=jnp.float32)
    nb = lax.dot_general(neigh, wn_ref[...], (((1,), (1,)), ((), ())),
                         preferred_element_type=jnp.float32)
    out_ref[...] = h + b_ref[...] + nb * (1.0 / deg)


_BM = 1024


@jax.jit
def kernel(feat, edge_index, W_neigh, W_self, b_self):
    padw = EPW - E // NW
    src2 = edge_index[0].reshape(NW, E // NW)
    dst2 = edge_index[1].reshape(NW, E // NW)
    lane = jnp.arange(padw, dtype=jnp.int32)[None, :]
    wrow = jnp.arange(NW, dtype=jnp.int32)[:, None]
    pad_src = (wrow * padw + lane) % N
    pad_dst = jnp.broadcast_to(N + lane, (NW, padw))
    src_p = jnp.concatenate([src2, pad_src], axis=1).reshape(NW, KCH, CH)
    dst_p = jnp.concatenate([dst2, pad_dst], axis=1).reshape(NW, KCH, CH)
    zeros = jnp.zeros((NPAD, D), jnp.float32)

    aggout, degout = _sc_aggregate(feat, src_p, dst_p, zeros)
    degout = degout.reshape(NW, NPAD)

    rst = pl.pallas_call(
        _combine_body,
        grid=(pl.cdiv(N, _BM),),
        in_specs=[
            pl.BlockSpec((_BM, D), lambda i: (i, 0)),
            pl.BlockSpec((NC, _BM, D), lambda i: (0, i, 0)),
            pl.BlockSpec((NW, _BM), lambda i: (0, i)),
            pl.BlockSpec((D, D), lambda i: (0, 0)),
            pl.BlockSpec((D, D), lambda i: (0, 0)),
            pl.BlockSpec((1, D), lambda i: (0, 0)),
        ],
        out_specs=pl.BlockSpec((_BM, D), lambda i: (i, 0)),
        out_shape=jax.ShapeDtypeStruct((N, D), jnp.float32),
    )(feat, aggout, degout, W_neigh, W_self, b_self.reshape(1, D))
    return rst


# speedup vs baseline: 1.8585x; 1.0675x over previous
import functools

import jax
import jax.numpy as jnp
from jax import lax
from jax.experimental import pallas as pl
from jax.experimental.pallas import tpu as pltpu, tpu_sc as plsc

N = 10000
E = 320000
D = 128
NC = 2
NS = 16
NW = NC * NS
CH = 128
KS = 8
ST = 10
KCH = KS * ST
EPW = KCH * CH
EPAD = NW * EPW
RPT = 640
NPAD = NS * RPT

_mesh = plsc.VectorSubcoreMesh(core_axis_name="c", subcore_axis_name="s")


@functools.partial(
    pl.kernel,
    out_type=(
        jax.ShapeDtypeStruct((NC, NPAD, D), jnp.float32),
        jax.ShapeDtypeStruct((NW, NPAD // 128, 128), jnp.float32),
    ),
    mesh=_mesh,
    compiler_params=pltpu.CompilerParams(needs_layout_passes=False),
    scratch_types=[
        pltpu.VMEM((KS, CH), jnp.int32),
        pltpu.VMEM((KS, CH), jnp.int32),
        pltpu.VMEM((KS, CH), jnp.int32),
        pltpu.VMEM((KS, CH), jnp.int32),
        pltpu.VMEM((CH, D), jnp.float32),
        pltpu.VMEM((CH, D), jnp.float32),
        pltpu.VMEM((NPAD // 128, 128), jnp.float32),
        pltpu.VMEM_SHARED((NPAD, D), jnp.float32),
        pltpu.SemaphoreType.DMA,
        pltpu.SemaphoreType.DMA,
        pltpu.SemaphoreType.DMA,
    ],
)
def _sc_aggregate(feat_hbm, srcv_hbm, dstv_hbm, zeros_hbm, out_hbm, deg_hbm,
                  src_a, dst_a, src_b, dst_b, rows0, rows1, deg_v, agg_sh,
                  sem0, sem1, sems):
    cid = lax.axis_index("c")
    sid = lax.axis_index("s")
    w = cid * NS + sid
    r0 = sid * RPT

    zero16 = jnp.zeros((16,), jnp.float32)
    one16 = jnp.ones((16,), jnp.float32)

    def zbody(i, carry):
        deg_v[i // 8, pl.ds((i % 8) * 16, 16)] = zero16
        return carry

    def _wait(sem, buf):
        pltpu.make_async_copy(feat_hbm.at[src_a.at[0]], buf, sem).wait()

    bufs = [(src_a, dst_a), (src_b, dst_b)]

    with jax.named_scope("stagecopy"):
        pltpu.sync_copy(srcv_hbm.at[w, pl.ds(0, KS)], src_a)
        pltpu.sync_copy(dstv_hbm.at[w, pl.ds(0, KS)], dst_a)
    pltpu.async_copy(feat_hbm.at[src_a.at[0]], rows0, sem0)
    pltpu.async_copy(feat_hbm.at[src_a.at[1]], rows1, sem1)

    with jax.named_scope("initzero"):
        pltpu.sync_copy(zeros_hbm.at[pl.ds(r0, RPT)], agg_sh.at[pl.ds(r0, RPT)])
        lax.fori_loop(0, NPAD // 16, zbody, 0)
    plsc.subcore_barrier()

    for st in range(ST):
        sv, dv = bufs[st % 2]
        nsv, ndv = bufs[(st + 1) % 2]
        last = st == ST - 1

        if not last:
            pltpu.async_copy(srcv_hbm.at[w, pl.ds((st + 1) * KS, KS)],
                             nsv, sems)
            pltpu.async_copy(dstv_hbm.at[w, pl.ds((st + 1) * KS, KS)],
                             ndv, sems)

        def dbody(i, carry):
            idx = dv[i // 8, pl.ds((i % 8) * 16, 16)]
            plsc.addupdate_scatter(deg_v, [idx >> 7, idx & 127], one16)
            return carry

        with jax.named_scope("degloop"):
            lax.fori_loop(0, (KS * CH) // 16, dbody, 0)

        def body(j, carry):
            c = 2 * j
            _wait(sem0, rows0)
            pltpu.sync_copy(rows0, agg_sh.at[dv.at[c]], add=True)
            pltpu.async_copy(feat_hbm.at[sv.at[c + 2]], rows0, sem0)
            _wait(sem1, rows1)
            pltpu.sync_copy(rows1, agg_sh.at[dv.at[c + 1]], add=True)
            pltpu.async_copy(feat_hbm.at[sv.at[c + 3]], rows1, sem1)
            return carry

        with jax.named_scope("gsloop"):
            lax.fori_loop(0, (KS - 2) // 2, body, 0)

        _wait(sem0, rows0)
        pltpu.sync_copy(rows0, agg_sh.at[dv.at[KS - 2]], add=True)
        if not last:
            pltpu.make_async_copy(srcv_hbm.at[w, pl.ds(0, KS)],
                                  nsv, sems).wait()
            pltpu.make_async_copy(srcv_hbm.at[w, pl.ds(0, KS)],
                                  ndv, sems).wait()
            pltpu.async_copy(feat_hbm.at[nsv.at[0]], rows0, sem0)
        _wait(sem1, rows1)
        pltpu.sync_copy(rows1, agg_sh.at[dv.at[KS - 1]], add=True)
        if not last:
            pltpu.async_copy(feat_hbm.at[nsv.at[1]], rows1, sem1)

    with jax.named_scope("writeback"):
        pltpu.sync_copy(deg_v, deg_hbm.at[w])

        plsc.subcore_barrier()
        pltpu.sync_copy(agg_sh.at[pl.ds(r0, RPT)],
                        out_hbm.at[cid, pl.ds(r0, RPT)])


def _combine_body(feat_ref, agg_ref, deg_ref, wn_ref, ws_ref, b_ref, out_ref):
    x = feat_ref[...]
    neigh = agg_ref[0] + agg_ref[1]
    deg = jnp.sum(deg_ref[...], axis=0)[:, None]
    h = lax.dot_general(x, ws_ref[...], (((1,), (1,)), ((), ())),
                        preferred_element_type=jnp.float32)
    nb = lax.dot_general(neigh, wn_ref[...], (((1,), (1,)), ((), ())),
                         preferred_element_type=jnp.float32)
    out_ref[...] = h + b_ref[...] + nb * (1.0 / deg)


_BM = 1024


@jax.jit
def kernel(feat, edge_index, W_neigh, W_self, b_self):
    padw = EPW - E // NW
    src2 = edge_index[0].reshape(NW, E // NW)
    dst2 = edge_index[1].reshape(NW, E // NW)
    lane = jnp.arange(padw, dtype=jnp.int32)[None, :]
    wrow = jnp.arange(NW, dtype=jnp.int32)[:, None]
    pad_src = (wrow * padw + lane) % N
    pad_dst = jnp.broadcast_to(N + lane, (NW, padw))
    src_p = jnp.concatenate([src2, pad_src], axis=1).reshape(NW, KCH, CH)
    dst_p = jnp.concatenate([dst2, pad_dst], axis=1).reshape(NW, KCH, CH)
    zeros = jnp.zeros((NPAD, D), jnp.float32)

    aggout, degout = _sc_aggregate(feat, src_p, dst_p, zeros)
    degout = degout.reshape(NW, NPAD)

    rst = pl.pallas_call(
        _combine_body,
        grid=(pl.cdiv(N, _BM),),
        in_specs=[
            pl.BlockSpec((_BM, D), lambda i: (i, 0)),
            pl.BlockSpec((NC, _BM, D), lambda i: (0, i, 0)),
            pl.BlockSpec((NW, _BM), lambda i: (0, i)),
            pl.BlockSpec((D, D), lambda i: (0, 0)),
            pl.BlockSpec((D, D), lambda i: (0, 0)),
            pl.BlockSpec((1, D), lambda i: (0, 0)),
        ],
        out_specs=pl.BlockSpec((_BM, D), lambda i: (i, 0)),
        out_shape=jax.ShapeDtypeStruct((N, D), jnp.float32),
    )(feat, aggout, degout, W_neigh, W_self, b_self.reshape(1, D))
    return rst


# speedup vs baseline: 1.8792x; 1.0112x over previous
import functools

import jax
import jax.numpy as jnp
from jax import lax
from jax.experimental import pallas as pl
from jax.experimental.pallas import tpu as pltpu, tpu_sc as plsc

N = 10000
E = 320000
D = 128
NC = 2
NS = 16
NW = NC * NS
CH = 128
KS = 8
ST = 10
KCH = KS * ST
EPW = KCH * CH
EPAD = NW * EPW
RPT = 640
NPAD = NS * RPT

_mesh = plsc.VectorSubcoreMesh(core_axis_name="c", subcore_axis_name="s")


@functools.partial(
    pl.kernel,
    out_type=(
        jax.ShapeDtypeStruct((NC, NPAD, D), jnp.float32),
        jax.ShapeDtypeStruct((NW, NPAD // 128, 128), jnp.float32),
    ),
    mesh=_mesh,
    compiler_params=pltpu.CompilerParams(needs_layout_passes=False),
    scratch_types=[
        pltpu.VMEM((KS, CH), jnp.int32),
        pltpu.VMEM((KS, CH), jnp.int32),
        pltpu.VMEM((KS, CH), jnp.int32),
        pltpu.VMEM((KS, CH), jnp.int32),
        pltpu.VMEM((CH, D), jnp.float32),
        pltpu.VMEM((CH, D), jnp.float32),
        pltpu.VMEM((NPAD // 128, 128), jnp.float32),
        pltpu.VMEM_SHARED((NPAD, D), jnp.float32),
        pltpu.SemaphoreType.DMA,
        pltpu.SemaphoreType.DMA,
        pltpu.SemaphoreType.DMA,
    ],
)
def _sc_aggregate(feat_hbm, srcv_hbm, dstv_hbm, zeros_hbm, out_hbm, deg_hbm,
                  src_a, dst_a, src_b, dst_b, rows0, rows1, deg_v, agg_sh,
                  sem0, sem1, sems):
    cid = lax.axis_index("c")
    sid = lax.axis_index("s")
    w = cid * NS + sid
    r0 = sid * RPT

    zero16 = jnp.zeros((16,), jnp.float32)
    one16 = jnp.ones((16,), jnp.float32)

    def zbody(i, carry):
        deg_v[i // 8, pl.ds((i % 8) * 16, 16)] = zero16
        return carry

    def _wait(sem, buf):
        pltpu.make_async_copy(feat_hbm.at[src_a.at[0]], buf, sem).wait()

    bufs = [(src_a, dst_a), (src_b, dst_b)]

    with jax.named_scope("stagecopy"):
        pltpu.sync_copy(srcv_hbm.at[w, pl.ds(0, KS)], src_a)
        pltpu.sync_copy(dstv_hbm.at[w, pl.ds(0, KS)], dst_a)
    pltpu.async_copy(feat_hbm.at[src_a.at[0]], rows0, sem0)
    pltpu.async_copy(feat_hbm.at[src_a.at[1]], rows1, sem1)

    with jax.named_scope("initzero"):
        pltpu.sync_copy(zeros_hbm.at[pl.ds(r0, RPT)], agg_sh.at[pl.ds(r0, RPT)])
        lax.fori_loop(0, NPAD // 16, zbody, 0)
    plsc.subcore_barrier()

    for st in range(ST):
        sv, dv = bufs[st % 2]
        nsv, ndv = bufs[(st + 1) % 2]
        last = st == ST - 1

        if not last:
            pltpu.async_copy(srcv_hbm.at[w, pl.ds((st + 1) * KS, KS)],
                             nsv, sems)
            pltpu.async_copy(dstv_hbm.at[w, pl.ds((st + 1) * KS, KS)],
                             ndv, sems)

        def dbody(i, carry):
            idx = dv[i // 8, pl.ds((i % 8) * 16, 16)]
            plsc.addupdate_scatter(deg_v, [idx >> 7, idx & 127], one16)
            return carry

        with jax.named_scope("degloop"):
            lax.fori_loop(0, (KS * CH) // 16, dbody, 0)

        def body(j, carry):
            c = 2 * j
            _wait(sem0, rows0)
            pltpu.sync_copy(rows0, agg_sh.at[dv.at[c]], add=True)
            pltpu.async_copy(feat_hbm.at[sv.at[c + 2]], rows0, sem0)
            _wait(sem1, rows1)
            pltpu.sync_copy(rows1, agg_sh.at[dv.at[c + 1]], add=True)
            pltpu.async_copy(feat_hbm.at[sv.at[c + 3]], rows1, sem1)
            return carry

        with jax.named_scope("gsloop"):
            lax.fori_loop(0, (KS - 2) // 2, body, 0)

        _wait(sem0, rows0)
        pltpu.sync_copy(rows0, agg_sh.at[dv.at[KS - 2]], add=True)
        if not last:
            pltpu.make_async_copy(srcv_hbm.at[w, pl.ds(0, KS)],
                                  nsv, sems).wait()
            pltpu.make_async_copy(srcv_hbm.at[w, pl.ds(0, KS)],
                                  ndv, sems).wait()
            pltpu.async_copy(feat_hbm.at[nsv.at[0]], rows0, sem0)
        _wait(sem1, rows1)
        pltpu.sync_copy(rows1, agg_sh.at[dv.at[KS - 1]], add=True)
        if not last:
            pltpu.async_copy(feat_hbm.at[nsv.at[1]], rows1, sem1)

    with jax.named_scope("writeback"):
        pltpu.sync_copy(deg_v, deg_hbm.at[w])

        plsc.subcore_barrier()
        pltpu.sync_copy(agg_sh.at[pl.ds(r0, RPT)],
                        out_hbm.at[cid, pl.ds(r0, RPT)])


def _combine_body(feat_ref, agg_ref, deg_ref, wn_ref, ws_ref, b_ref, out_ref):
    x = feat_ref[...]
    neigh = agg_ref[0] + agg_ref[1]
    deg = jnp.sum(deg_ref[...], axis=0)[:, None]
    h = lax.dot_general(x, ws_ref[...], (((1,), (1,)), ((), ())),
                        preferred_element_type=jnp.float32)
    nb = lax.dot_general(neigh, wn_ref[...], (((1,), (1,)), ((), ())),
                         preferred_element_type=jnp.float32)
    out_ref[...] = h + b_ref[...] + nb * (1.0 / deg)


_BM = 2048


@jax.jit
def kernel(feat, edge_index, W_neigh, W_self, b_self):
    padw = EPW - E // NW
    src2 = edge_index[0].reshape(NW, E // NW)
    dst2 = edge_index[1].reshape(NW, E // NW)
    lane = jnp.arange(padw, dtype=jnp.int32)[None, :]
    wrow = jnp.arange(NW, dtype=jnp.int32)[:, None]
    pad_src = (wrow * padw + lane) % N
    pad_dst = jnp.broadcast_to(N + lane, (NW, padw))
    src_p = jnp.concatenate([src2, pad_src], axis=1).reshape(NW, KCH, CH)
    dst_p = jnp.concatenate([dst2, pad_dst], axis=1).reshape(NW, KCH, CH)
    zeros = jnp.zeros((NPAD, D), jnp.float32)

    aggout, degout = _sc_aggregate(feat, src_p, dst_p, zeros)
    degout = degout.reshape(NW, NPAD)

    rst = pl.pallas_call(
        _combine_body,
        grid=(pl.cdiv(N, _BM),),
        in_specs=[
            pl.BlockSpec((_BM, D), lambda i: (i, 0)),
            pl.BlockSpec((NC, _BM, D), lambda i: (0, i, 0)),
            pl.BlockSpec((NW, _BM), lambda i: (0, i)),
            pl.BlockSpec((D, D), lambda i: (0, 0)),
            pl.BlockSpec((D, D), lambda i: (0, 0)),
            pl.BlockSpec((1, D), lambda i: (0, 0)),
        ],
        out_specs=pl.BlockSpec((_BM, D), lambda i: (i, 0)),
        out_shape=jax.ShapeDtypeStruct((N, D), jnp.float32),
    )(feat, aggout, degout, W_neigh, W_self, b_self.reshape(1, D))
    return rst


# speedup vs baseline: 1.9581x; 1.0420x over previous
import functools

import jax
import jax.numpy as jnp
from jax import lax
from jax.experimental import pallas as pl
from jax.experimental.pallas import tpu as pltpu, tpu_sc as plsc

N = 10000
E = 320000
D = 128
NC = 2
NS = 16
NW = NC * NS
CH = 128
KS = 8
ST = 10
KCH = KS * ST
EPW = KCH * CH
EPAD = NW * EPW
RPT = 640
NPAD = NS * RPT

_mesh = plsc.VectorSubcoreMesh(core_axis_name="c", subcore_axis_name="s")


@functools.partial(
    pl.kernel,
    out_type=(
        jax.ShapeDtypeStruct((NC, NPAD, D), jnp.float32),
        jax.ShapeDtypeStruct((NW, NPAD // 128, 128), jnp.float32),
    ),
    mesh=_mesh,
    compiler_params=pltpu.CompilerParams(needs_layout_passes=False),
    scratch_types=[
        pltpu.VMEM((KS, CH), jnp.int32),
        pltpu.VMEM((KS, CH), jnp.int32),
        pltpu.VMEM((KS, CH), jnp.int32),
        pltpu.VMEM((KS, CH), jnp.int32),
        pltpu.VMEM((CH, D), jnp.float32),
        pltpu.VMEM((CH, D), jnp.float32),
        pltpu.VMEM((NPAD // 128, 128), jnp.float32),
        pltpu.VMEM_SHARED((NPAD, D), jnp.float32),
        pltpu.SemaphoreType.DMA,
        pltpu.SemaphoreType.DMA,
        pltpu.SemaphoreType.DMA,
    ],
)
def _sc_aggregate(feat_hbm, srcv_hbm, dstv_hbm, out_hbm, deg_hbm,
                  src_a, dst_a, src_b, dst_b, rows0, rows1, deg_v, agg_sh,
                  sem0, sem1, sems):
    cid = lax.axis_index("c")
    sid = lax.axis_index("s")
    w = cid * NS + sid
    r0 = sid * RPT

    zero16 = jnp.zeros((16,), jnp.float32)
    one16 = jnp.ones((16,), jnp.float32)

    def zbody(i, carry):
        deg_v[i // 8, pl.ds((i % 8) * 16, 16)] = zero16
        return carry

    def _wait(sem, buf):
        pltpu.make_async_copy(feat_hbm.at[src_a.at[0]], buf, sem).wait()

    bufs = [(src_a, dst_a), (src_b, dst_b)]

    with jax.named_scope("stagecopy"):
        pltpu.sync_copy(srcv_hbm.at[w, pl.ds(0, KS)], src_a)
        pltpu.sync_copy(dstv_hbm.at[w, pl.ds(0, KS)], dst_a)
    pltpu.async_copy(feat_hbm.at[src_a.at[0]], rows0, sem0)
    pltpu.async_copy(feat_hbm.at[src_a.at[1]], rows1, sem1)

    DROWS = NPAD // 128
    with jax.named_scope("initzero"):
        lax.fori_loop(0, NPAD // 16, zbody, 0)
        for i in range(RPT // DROWS):
            pltpu.async_copy(deg_v, agg_sh.at[pl.ds(r0 + i * DROWS, DROWS)],
                             sems)
        for i in range(RPT // DROWS):
            pltpu.make_async_copy(deg_v, agg_sh.at[pl.ds(r0, DROWS)],
                                  sems).wait()
    plsc.subcore_barrier()

    for st in range(ST):
        sv, dv = bufs[st % 2]
        nsv, ndv = bufs[(st + 1) % 2]
        last = st == ST - 1

        if not last:
            pltpu.async_copy(srcv_hbm.at[w, pl.ds((st + 1) * KS, KS)],
                             nsv, sems)
            pltpu.async_copy(dstv_hbm.at[w, pl.ds((st + 1) * KS, KS)],
                             ndv, sems)

        def dbody(i, carry):
            idx = dv[i // 8, pl.ds((i % 8) * 16, 16)]
            plsc.addupdate_scatter(deg_v, [idx >> 7, idx & 127], one16)
            return carry

        with jax.named_scope("degloop"):
            lax.fori_loop(0, (KS * CH) // 16, dbody, 0)

        def body(j, carry):
            c = 2 * j
            _wait(sem0, rows0)
            pltpu.sync_copy(rows0, agg_sh.at[dv.at[c]], add=True)
            pltpu.async_copy(feat_hbm.at[sv.at[c + 2]], rows0, sem0)
            _wait(sem1, rows1)
            pltpu.sync_copy(rows1, agg_sh.at[dv.at[c + 1]], add=True)
            pltpu.async_copy(feat_hbm.at[sv.at[c + 3]], rows1, sem1)
            return carry

        with jax.named_scope("gsloop"):
            lax.fori_loop(0, (KS - 2) // 2, body, 0)

        _wait(sem0, rows0)
        pltpu.sync_copy(rows0, agg_sh.at[dv.at[KS - 2]], add=True)
        if not last:
            pltpu.make_async_copy(srcv_hbm.at[w, pl.ds(0, KS)],
                                  nsv, sems).wait()
            pltpu.make_async_copy(srcv_hbm.at[w, pl.ds(0, KS)],
                                  ndv, sems).wait()
            pltpu.async_copy(feat_hbm.at[nsv.at[0]], rows0, sem0)
        _wait(sem1, rows1)
        pltpu.sync_copy(rows1, agg_sh.at[dv.at[KS - 1]], add=True)
        if not last:
            pltpu.async_copy(feat_hbm.at[nsv.at[1]], rows1, sem1)

    with jax.named_scope("writeback"):
        pltpu.sync_copy(deg_v, deg_hbm.at[w])

        plsc.subcore_barrier()
        pltpu.sync_copy(agg_sh.at[pl.ds(r0, RPT)],
                        out_hbm.at[cid, pl.ds(r0, RPT)])


def _combine_body(feat_ref, agg_ref, deg_ref, wn_ref, ws_ref, b_ref, out_ref):
    x = feat_ref[...]
    neigh = agg_ref[0] + agg_ref[1]
    deg = jnp.sum(deg_ref[...], axis=0)[:, None]
    h = lax.dot_general(x, ws_ref[...], (((1,), (1,)), ((), ())),
                        preferred_element_type=jnp.float32)
    nb = lax.dot_general(neigh, wn_ref[...], (((1,), (1,)), ((), ())),
                         preferred_element_type=jnp.float32)
    out_ref[...] = h + b_ref[...] + nb * (1.0 / deg)


_BM = 2048


@jax.jit
def kernel(feat, edge_index, W_neigh, W_self, b_self):
    padw = EPW - E // NW
    src2 = edge_index[0].reshape(NW, E // NW)
    dst2 = edge_index[1].reshape(NW, E // NW)
    lane = jnp.arange(padw, dtype=jnp.int32)[None, :]
    wrow = jnp.arange(NW, dtype=jnp.int32)[:, None]
    pad_src = (wrow * padw + lane) % N
    pad_dst = jnp.broadcast_to(N + lane, (NW, padw))
    src_p = jnp.concatenate([src2, pad_src], axis=1).reshape(NW, KCH, CH)
    dst_p = jnp.concatenate([dst2, pad_dst], axis=1).reshape(NW, KCH, CH)

    aggout, degout = _sc_aggregate(feat, src_p, dst_p)
    degout = degout.reshape(NW, NPAD)

    rst = pl.pallas_call(
        _combine_body,
        grid=(pl.cdiv(N, _BM),),
        in_specs=[
            pl.BlockSpec((_BM, D), lambda i: (i, 0)),
            pl.BlockSpec((NC, _BM, D), lambda i: (0, i, 0)),
            pl.BlockSpec((NW, _BM), lambda i: (0, i)),
            pl.BlockSpec((D, D), lambda i: (0, 0)),
            pl.BlockSpec((D, D), lambda i: (0, 0)),
            pl.BlockSpec((1, D), lambda i: (0, 0)),
        ],
        out_specs=pl.BlockSpec((_BM, D), lambda i: (i, 0)),
        out_shape=jax.ShapeDtypeStruct((N, D), jnp.float32),
    )(feat, aggout, degout, W_neigh, W_self, b_self.reshape(1, D))
    return rst
